# baseline (speedup 1.0000x reference)
import functools
import math

import jax
import jax.numpy as jnp
import numpy as np
from jax import lax
from jax.experimental import pallas as pl
from jax.experimental.pallas import tpu as pltpu
from jax.experimental.pallas import tpu_sc as plsc

NH, NL, NP = 8, 4, 8
D = 32
LQ = 1024
C = 256
LEN_IN = 5440
LSI = (0, 4096, 5120, 5376)
WL = (64, 32, 16, 8)
NB = 2


_RB = 544


def _value_body(src_ref, g_ref, b_ref, wv_ref, bv_ref, out_ref):
    s = src_ref[0]
    mu = jnp.mean(s, axis=-1, keepdims=True)
    var = jnp.mean((s - mu) ** 2, axis=-1, keepdims=True)
    sn = (s - mu) / jnp.sqrt(var + 1e-5) * g_ref[0] + b_ref[0]
    val = lax.dot_general(
        sn, wv_ref[...], (((1,), (1,)), ((), ())),
        preferred_element_type=jnp.float32) + bv_ref[0]
    out_ref[0] = val.astype(jnp.bfloat16)


def _value_call(src, g, b, wv, bv):
    return pl.pallas_call(
        _value_body,
        grid=(NB, LEN_IN // _RB),
        in_specs=[
            pl.BlockSpec((1, _RB, C), lambda n, r: (n, r, 0)),
            pl.BlockSpec((1, C), lambda n, r: (0, 0)),
            pl.BlockSpec((1, C), lambda n, r: (0, 0)),
            pl.BlockSpec((C, C), lambda n, r: (0, 0)),
            pl.BlockSpec((1, C), lambda n, r: (0, 0)),
        ],
        out_specs=pl.BlockSpec((1, _RB, C), lambda n, r: (n, r, 0)),
        out_shape=jax.ShapeDtypeStruct((NB, LEN_IN, C), jnp.bfloat16),
    )(src, g.reshape(1, C), b.reshape(1, C), wv, bv.reshape(1, C))


def _tap_body(tgt_ref, rx_ref, ry_ref, wsox_ref, wsoy_ref, waw_ref,
              bsox_ref, bsoy_ref, baw_ref, g_ref, b_ref, idx_ref, w_ref):
    t = tgt_ref[0]
    mu = jnp.mean(t, axis=-1, keepdims=True)
    var = jnp.mean((t - mu) ** 2, axis=-1, keepdims=True)
    tn = (t - mu) / jnp.sqrt(var + 1e-5) * g_ref[0] + b_ref[0]

    def dot(a, w):
        return lax.dot_general(a, w, (((1,), (1,)), ((), ())),
                               preferred_element_type=jnp.float32)

    sox = dot(tn, wsox_ref[...]) + bsox_ref[0]
    soy = dot(tn, wsoy_ref[...]) + bsoy_ref[0]
    logits = dot(tn, waw_ref[...]) + baw_ref[0]
    aw_parts = []
    for h in range(NH):
        sl = logits[:, h * 32:(h + 1) * 32]
        m = jnp.max(sl, axis=-1, keepdims=True)
        e = jnp.exp(sl - m)
        aw_parts.append(e / jnp.sum(e, axis=-1, keepdims=True))
    aw = jnp.concatenate(aw_parts, axis=-1)

    lane = lax.broadcasted_iota(jnp.int32, (LQ, NH * NL * NP), 1)
    lvl = (lane % (NL * NP)) // NP
    wl_i = jnp.where(lvl == 0, WL[0],
                     jnp.where(lvl == 1, WL[1],
                               jnp.where(lvl == 2, WL[2], WL[3])))
    lsi = jnp.where(lvl == 0, LSI[0],
                    jnp.where(lvl == 1, LSI[1],
                              jnp.where(lvl == 2, LSI[2], LSI[3])))
    wl_f = wl_i.astype(jnp.float32)

    locx = rx_ref[0] + sox / wl_f
    locy = ry_ref[0] + soy / wl_f
    px = (2.0 * locx - 1.0 + 1.0) * 0.5 * wl_f - 0.5
    py = (2.0 * locy - 1.0 + 1.0) * 0.5 * wl_f - 0.5
    x0f = jnp.floor(px)
    y0f = jnp.floor(py)
    fx = px - x0f
    fy = py - y0f
    x0 = x0f.astype(jnp.int32)
    y0 = y0f.astype(jnp.int32)
    x1 = x0 + 1
    y1 = y0 + 1
    vx0 = (x0 >= 0) & (x0 < wl_i)
    vx1 = (x1 >= 0) & (x1 < wl_i)
    vy0 = (y0 >= 0) & (y0 < wl_i)
    vy1 = (y1 >= 0) & (y1 < wl_i)
    x0c = jnp.clip(x0, 0, wl_i - 1)
    x1c = jnp.clip(x1, 0, wl_i - 1)
    y0c = jnp.clip(y0, 0, wl_i - 1)
    y1c = jnp.clip(y1, 0, wl_i - 1)
    hh = lane // (NL * NP)

    def mkidx(yc, xc):
        return (lsi + yc * wl_i + xc) * NH + hh

    def mkw(wx, wy, v):
        return wx * wy * aw * v.astype(jnp.float32)

    i00 = mkidx(y0c, x0c)
    i01 = mkidx(y0c, x1c)
    i10 = mkidx(y1c, x0c)
    i11 = mkidx(y1c, x1c)
    w00 = mkw(1.0 - fx, 1.0 - fy, vx0 & vy0)
    w01 = mkw(fx, 1.0 - fy, vx1 & vy0)
    w10 = mkw(1.0 - fx, fy, vx0 & vy1)
    w11 = mkw(fx, fy, vx1 & vy1)

    def dup_pack(w):
        u = lax.bitcast_convert_type(w.astype(jnp.bfloat16),
                                     jnp.uint16).astype(jnp.uint32)
        return (u | (u << 16)).astype(jnp.int32)

    for h in range(NH):
        sl = slice(h * 32, (h + 1) * 32)
        idx_ref[0, h] = jnp.concatenate(
            [i00[:, sl], i01[:, sl], i10[:, sl], i11[:, sl]], axis=-1)
        w_ref[0, h] = dup_pack(jnp.concatenate(
            [w00[:, sl], w01[:, sl], w10[:, sl], w11[:, sl]], axis=-1))


def _tap_call(tgt, rx, ry, wsox, wsoy, waw, bsox, bsoy, baw, g, b):
    return pl.pallas_call(
        _tap_body,
        grid=(NB,),
        in_specs=[
            pl.BlockSpec((1, LQ, C), lambda n: (n, 0, 0)),
            pl.BlockSpec((1, LQ, NH * NL * NP), lambda n: (n, 0, 0)),
            pl.BlockSpec((1, LQ, NH * NL * NP), lambda n: (n, 0, 0)),
            pl.BlockSpec((NH * NL * NP, C), lambda n: (0, 0)),
            pl.BlockSpec((NH * NL * NP, C), lambda n: (0, 0)),
            pl.BlockSpec((NH * NL * NP, C), lambda n: (0, 0)),
            pl.BlockSpec((1, NH * NL * NP), lambda n: (0, 0)),
            pl.BlockSpec((1, NH * NL * NP), lambda n: (0, 0)),
            pl.BlockSpec((1, NH * NL * NP), lambda n: (0, 0)),
            pl.BlockSpec((1, C), lambda n: (0, 0)),
            pl.BlockSpec((1, C), lambda n: (0, 0)),
        ],
        out_specs=[
            pl.BlockSpec((1, NH, LQ, 4 * NL * NP), lambda n: (n, 0, 0, 0)),
            pl.BlockSpec((1, NH, LQ, 4 * NL * NP), lambda n: (n, 0, 0, 0)),
        ],
        out_shape=[
            jax.ShapeDtypeStruct((NB, NH, LQ, 4 * NL * NP), jnp.int32),
            jax.ShapeDtypeStruct((NB, NH, LQ, 4 * NL * NP), jnp.int32),
        ],
    )(tgt, rx, ry, wsox, wsoy, waw, bsox, bsoy, baw,
      g.reshape(1, C), b.reshape(1, C))


_NC, _NS = 2, 16
_NW = _NC * _NS
_O = NB * NH * LQ
_OPW = _O // _NW
_CH = 16
_NCHUNK = _OPW // _CH
_NT = 4 * NL * NP


def _sc_body(val_hbm, idx_hbm, w_hbm, out_hbm, val_sh, idx_v, w_v, rows_v,
             out_v, sem_g, sem_t):
    cid = lax.axis_index("c")
    wid = cid * _NS + lax.axis_index("s")
    base = wid * _OPW
    cbase = wid * _NCHUNK

    def fire_taps(ci, tb):
        row = cbase + ci
        pltpu.async_copy(idx_hbm.at[pl.ds(row, 1)], idx_v.at[tb], sem_t)
        pltpu.async_copy(w_hbm.at[pl.ds(row, 1)], w_v.at[tb], sem_t)

    def drain_taps(tb):
        pltpu.make_async_copy(
            idx_hbm.at[pl.ds(0, 1)], idx_v.at[tb], sem_t).wait()
        pltpu.make_async_copy(
            w_hbm.at[pl.ds(0, 1)], w_v.at[tb], sem_t).wait()

    def fire_gathers(tb, rb):
        for j in range(_CH):
            pltpu.async_copy(
                val_sh.at[idx_v.at[tb].at[0].at[pl.ds(j * _NT, _NT)]],
                rows_v.at[rb].at[pl.ds(j * _NT, _NT)], sem_g)

    @pl.when(lax.axis_index("s") == 0)
    def _():
        pltpu.sync_copy(val_hbm.at[pl.ds(cid * (LEN_IN * NH), LEN_IN * NH)],
                        val_sh)
    plsc.subcore_barrier()

    pltpu.sync_copy(idx_hbm.at[pl.ds(cbase, 1)], idx_v.at[0])
    pltpu.sync_copy(w_hbm.at[pl.ds(cbase, 1)], w_v.at[0])
    fire_gathers(0, 0)
    fire_taps(1, 1)

    def chunk(c, carry):
        rb = lax.rem(c, 2)
        tb_c = lax.rem(c, 3)
        tb_n = lax.rem(c + 1, 3)
        tb_p = lax.rem(c + 2, 3)
        pltpu.make_async_copy(
            val_hbm.at[pl.ds(0, _CH * _NT)], rows_v.at[rb], sem_g).wait()

        @pl.when(c + 1 < _NCHUNK)
        def _():
            drain_taps(tb_n)
            fire_gathers(tb_n, 1 - rb)

        for j in range(_CH):
            def grp(g, carry):
                m0, m1 = carry
                wvp = w_v[tb_c, 0, pl.ds(j * _NT + g * 16, 16)]
                r0 = j * _NT + g * 16
                b = [jnp.zeros((32,), jnp.bfloat16) for _ in range(4)]
                for k in range(16):
                    wt = plsc.bitcast(
                        jnp.full((16,), wvp[k], dtype=jnp.int32),
                        jnp.bfloat16)
                    b[k % 4] = b[k % 4] + wt * rows_v[rb, r0 + k]
                s = (b[0] + b[1]) + (b[2] + b[3])
                e, o = plsc.unpack(s, format=plsc.PackFormat.INTERLEAVED)
                return (m0 + e, m1 + o)

            m0, m1 = lax.fori_loop(
                0, _NT // 16, grp,
                (jnp.zeros((16,), jnp.float32), jnp.zeros((16,), jnp.float32)),
                unroll=2)
            out_v[j, pl.ds(0, 16)] = m0
            out_v[j, pl.ds(16, 16)] = m1
        pltpu.sync_copy(out_v, out_hbm.at[pl.ds(base + c * _CH, _CH)])

        @pl.when(c + 2 < _NCHUNK)
        def _():
            fire_taps(c + 2, tb_p)

        return carry

    lax.fori_loop(0, _NCHUNK, chunk, 0)


@functools.partial(jax.jit)
def _sc_gather(val_flat, idx, w):
    mesh = plsc.VectorSubcoreMesh(core_axis_name="c", subcore_axis_name="s")
    return pl.kernel(
        _sc_body,
        out_type=jax.ShapeDtypeStruct((_O, D), jnp.float32),
        mesh=mesh,
        scratch_types=[
            pltpu.VMEM_SHARED((LEN_IN * NH, D), jnp.bfloat16),
            pltpu.VMEM((3, 1, _CH * _NT), jnp.int32),
            pltpu.VMEM((3, 1, _CH * _NT), jnp.int32),
            pltpu.VMEM((2, _CH * _NT, D), jnp.bfloat16),
            pltpu.VMEM((_CH, D), jnp.float32),
            pltpu.SemaphoreType.DMA,
            pltpu.SemaphoreType.DMA,
        ],
        compiler_params=pltpu.CompilerParams(use_tc_tiling_on_sc=False,
                                             needs_layout_passes=False),
    )(val_flat, idx, w)


def _out_body(core_ref, tgt_ref, wo_ref, bo_ref, g_ref, b_ref,
              wfc1_ref, bfc1_ref, wfc2_ref, bfc2_ref, out_ref):
    def dot(a, w):
        return lax.dot_general(a, w, (((1,), (1,)), ((), ())),
                               preferred_element_type=jnp.float32)

    acc = jnp.broadcast_to(bo_ref[0], (LQ, C))
    for hh in range(NH):
        acc = acc + dot(core_ref[0, hh], wo_ref[:, hh * D:(hh + 1) * D])
    tgt2 = tgt_ref[0] + acc
    mu = jnp.mean(tgt2, axis=-1, keepdims=True)
    var = jnp.mean((tgt2 - mu) ** 2, axis=-1, keepdims=True)
    tn = (tgt2 - mu) / jnp.sqrt(var + 1e-5) * g_ref[0] + b_ref[0]
    hm = dot(tn, wfc1_ref[...]) + bfc1_ref[0]
    hm = 0.5 * hm * (1.0 + lax.erf(hm * (1.0 / math.sqrt(2.0))))
    out_ref[0] = tgt2 + dot(hm, wfc2_ref[...]) + bfc2_ref[0]


def _out_call(core, tgt, wo, bo, g, b, wfc1, bfc1, wfc2, bfc2):
    return pl.pallas_call(
        _out_body,
        grid=(NB,),
        in_specs=[
            pl.BlockSpec((1, NH, LQ, D), lambda n: (n, 0, 0, 0)),
            pl.BlockSpec((1, LQ, C), lambda n: (n, 0, 0)),
            pl.BlockSpec((C, C), lambda n: (0, 0)),
            pl.BlockSpec((1, C), lambda n: (0, 0)),
            pl.BlockSpec((1, C), lambda n: (0, 0)),
            pl.BlockSpec((1, C), lambda n: (0, 0)),
            pl.BlockSpec((4 * C, C), lambda n: (0, 0)),
            pl.BlockSpec((1, 4 * C), lambda n: (0, 0)),
            pl.BlockSpec((C, 4 * C), lambda n: (0, 0)),
            pl.BlockSpec((1, C), lambda n: (0, 0)),
        ],
        out_specs=pl.BlockSpec((1, LQ, C), lambda n: (n, 0, 0)),
        out_shape=jax.ShapeDtypeStruct((NB, LQ, C), jnp.float32),
    )(core, tgt, wo, bo.reshape(1, C), g.reshape(1, C), b.reshape(1, C),
      wfc1, bfc1.reshape(1, 4 * C), wfc2, bfc2.reshape(1, C))


def kernel(tgt, reference_points, src, src_spatial_shapes, level_start_index,
           norm1_g, norm1_b, norm2_g, norm2_b, W_so, b_so, W_aw, b_aw,
           W_v, b_v, W_o, b_o, W_fc1, b_fc1, W_fc2, b_fc2):
    value = _value_call(src, norm1_g, norm1_b, W_v, b_v)
    val_flat = value.reshape(NB * LEN_IN * NH, D)

    rx = jnp.broadcast_to(reference_points[:, :, None, :, None, 0],
                          (NB, LQ, NH, NL, NP)).reshape(NB, LQ, NH * NL * NP)
    ry = jnp.broadcast_to(reference_points[:, :, None, :, None, 1],
                          (NB, LQ, NH, NL, NP)).reshape(NB, LQ, NH * NL * NP)
    wso_r = W_so.reshape(NH, NL, NP, 2, C)
    bso_r = b_so.reshape(NH, NL, NP, 2)
    wsox = wso_r[:, :, :, 0, :].reshape(NH * NL * NP, C)
    wsoy = wso_r[:, :, :, 1, :].reshape(NH * NL * NP, C)
    bsox = bso_r[:, :, :, 0].reshape(1, NH * NL * NP)
    bsoy = bso_r[:, :, :, 1].reshape(1, NH * NL * NP)
    waw = W_aw.reshape(NH * NL * NP, C)
    baw = b_aw.reshape(1, NH * NL * NP)

    idx, w = _tap_call(tgt, rx, ry, wsox, wsoy, waw, bsox, bsoy, baw,
                       norm1_g, norm1_b)
    core = _sc_gather(val_flat, idx.reshape(_O // _CH, _CH * _NT),
                      w.reshape(_O // _CH, _CH * _NT))
    core4 = core.reshape(NB, NH, LQ, D)
    wo_perm = W_o.reshape(C, NH, D // 2, 2).transpose(0, 1, 3, 2).reshape(C, C)
    return _out_call(core4, tgt, wo_perm, b_o, norm2_g, norm2_b,
                     W_fc1, b_fc1, W_fc2, b_fc2)


# speedup vs baseline: 1.1342x; 1.1342x over previous
import functools
import math

import jax
import jax.numpy as jnp
import numpy as np
from jax import lax
from jax.experimental import pallas as pl
from jax.experimental.pallas import tpu as pltpu
from jax.experimental.pallas import tpu_sc as plsc

NH, NL, NP = 8, 4, 8
D = 32
LQ = 1024
C = 256
LEN_IN = 5440
LSI = (0, 4096, 5120, 5376)
WL = (64, 32, 16, 8)
NB = 2


_RB = 544


def _value_body(src_ref, g_ref, b_ref, wv_ref, bv_ref, out_ref):
    s = src_ref[0]
    mu = jnp.mean(s, axis=-1, keepdims=True)
    var = jnp.mean((s - mu) ** 2, axis=-1, keepdims=True)
    sn = (s - mu) / jnp.sqrt(var + 1e-5) * g_ref[0] + b_ref[0]
    val = lax.dot_general(
        sn, wv_ref[...], (((1,), (1,)), ((), ())),
        preferred_element_type=jnp.float32) + bv_ref[0]
    out_ref[0] = val.astype(jnp.bfloat16)


def _value_call(src, g, b, wv, bv):
    return pl.pallas_call(
        _value_body,
        grid=(NB, LEN_IN // _RB),
        in_specs=[
            pl.BlockSpec((1, _RB, C), lambda n, r: (n, r, 0)),
            pl.BlockSpec((1, C), lambda n, r: (0, 0)),
            pl.BlockSpec((1, C), lambda n, r: (0, 0)),
            pl.BlockSpec((C, C), lambda n, r: (0, 0)),
            pl.BlockSpec((1, C), lambda n, r: (0, 0)),
        ],
        out_specs=pl.BlockSpec((1, _RB, C), lambda n, r: (n, r, 0)),
        out_shape=jax.ShapeDtypeStruct((NB, LEN_IN, C), jnp.bfloat16),
    )(src, g.reshape(1, C), b.reshape(1, C), wv, bv.reshape(1, C))


def _tap_body(tgt_ref, rx_ref, ry_ref, wsox_ref, wsoy_ref, waw_ref,
              bsox_ref, bsoy_ref, baw_ref, g_ref, b_ref, idx_ref, w_ref):
    t = tgt_ref[0]
    mu = jnp.mean(t, axis=-1, keepdims=True)
    var = jnp.mean((t - mu) ** 2, axis=-1, keepdims=True)
    tn = (t - mu) / jnp.sqrt(var + 1e-5) * g_ref[0] + b_ref[0]

    def dot(a, w):
        return lax.dot_general(a, w, (((1,), (1,)), ((), ())),
                               preferred_element_type=jnp.float32)

    sox = dot(tn, wsox_ref[...]) + bsox_ref[0]
    soy = dot(tn, wsoy_ref[...]) + bsoy_ref[0]
    logits = dot(tn, waw_ref[...]) + baw_ref[0]
    aw_parts = []
    for h in range(NH):
        sl = logits[:, h * 32:(h + 1) * 32]
        m = jnp.max(sl, axis=-1, keepdims=True)
        e = jnp.exp(sl - m)
        aw_parts.append(e / jnp.sum(e, axis=-1, keepdims=True))
    aw = jnp.concatenate(aw_parts, axis=-1)

    lane = lax.broadcasted_iota(jnp.int32, (LQ, NH * NL * NP), 1)
    lvl = (lane % (NL * NP)) // NP
    wl_i = jnp.where(lvl == 0, WL[0],
                     jnp.where(lvl == 1, WL[1],
                               jnp.where(lvl == 2, WL[2], WL[3])))
    lsi = jnp.where(lvl == 0, LSI[0],
                    jnp.where(lvl == 1, LSI[1],
                              jnp.where(lvl == 2, LSI[2], LSI[3])))
    wl_f = wl_i.astype(jnp.float32)

    locx = rx_ref[0] + sox / wl_f
    locy = ry_ref[0] + soy / wl_f
    px = (2.0 * locx - 1.0 + 1.0) * 0.5 * wl_f - 0.5
    py = (2.0 * locy - 1.0 + 1.0) * 0.5 * wl_f - 0.5
    x0f = jnp.floor(px)
    y0f = jnp.floor(py)
    fx = px - x0f
    fy = py - y0f
    x0 = x0f.astype(jnp.int32)
    y0 = y0f.astype(jnp.int32)
    x1 = x0 + 1
    y1 = y0 + 1
    vx0 = (x0 >= 0) & (x0 < wl_i)
    vx1 = (x1 >= 0) & (x1 < wl_i)
    vy0 = (y0 >= 0) & (y0 < wl_i)
    vy1 = (y1 >= 0) & (y1 < wl_i)
    x0c = jnp.clip(x0, 0, wl_i - 1)
    x1c = jnp.clip(x1, 0, wl_i - 1)
    y0c = jnp.clip(y0, 0, wl_i - 1)
    y1c = jnp.clip(y1, 0, wl_i - 1)
    hh = lane // (NL * NP)

    def mkidx(yc, xc):
        return (lsi + yc * wl_i + xc) * NH + hh

    def mkw(wx, wy, v):
        return wx * wy * aw * v.astype(jnp.float32)

    i00 = mkidx(y0c, x0c)
    i01 = mkidx(y0c, x1c)
    i10 = mkidx(y1c, x0c)
    i11 = mkidx(y1c, x1c)
    w00 = mkw(1.0 - fx, 1.0 - fy, vx0 & vy0)
    w01 = mkw(fx, 1.0 - fy, vx1 & vy0)
    w10 = mkw(1.0 - fx, fy, vx0 & vy1)
    w11 = mkw(fx, fy, vx1 & vy1)

    def dup_pack(w):
        u = lax.bitcast_convert_type(w.astype(jnp.bfloat16),
                                     jnp.uint16).astype(jnp.uint32)
        return (u | (u << 16)).astype(jnp.int32)

    for h in range(NH):
        sl = slice(h * 32, (h + 1) * 32)
        idx_ref[0, h] = jnp.concatenate(
            [i00[:, sl], i01[:, sl], i10[:, sl], i11[:, sl]], axis=-1)
        w_ref[0, h] = dup_pack(jnp.concatenate(
            [w00[:, sl], w01[:, sl], w10[:, sl], w11[:, sl]], axis=-1))


def _tap_call(tgt, rx, ry, wsox, wsoy, waw, bsox, bsoy, baw, g, b):
    return pl.pallas_call(
        _tap_body,
        grid=(NB,),
        in_specs=[
            pl.BlockSpec((1, LQ, C), lambda n: (n, 0, 0)),
            pl.BlockSpec((1, LQ, NH * NL * NP), lambda n: (n, 0, 0)),
            pl.BlockSpec((1, LQ, NH * NL * NP), lambda n: (n, 0, 0)),
            pl.BlockSpec((NH * NL * NP, C), lambda n: (0, 0)),
            pl.BlockSpec((NH * NL * NP, C), lambda n: (0, 0)),
            pl.BlockSpec((NH * NL * NP, C), lambda n: (0, 0)),
            pl.BlockSpec((1, NH * NL * NP), lambda n: (0, 0)),
            pl.BlockSpec((1, NH * NL * NP), lambda n: (0, 0)),
            pl.BlockSpec((1, NH * NL * NP), lambda n: (0, 0)),
            pl.BlockSpec((1, C), lambda n: (0, 0)),
            pl.BlockSpec((1, C), lambda n: (0, 0)),
        ],
        out_specs=[
            pl.BlockSpec((1, NH, LQ, 4 * NL * NP), lambda n: (n, 0, 0, 0)),
            pl.BlockSpec((1, NH, LQ, 4 * NL * NP), lambda n: (n, 0, 0, 0)),
        ],
        out_shape=[
            jax.ShapeDtypeStruct((NB, NH, LQ, 4 * NL * NP), jnp.int32),
            jax.ShapeDtypeStruct((NB, NH, LQ, 4 * NL * NP), jnp.int32),
        ],
    )(tgt, rx, ry, wsox, wsoy, waw, bsox, bsoy, baw,
      g.reshape(1, C), b.reshape(1, C))


_NC, _NS = 2, 16
_NW = _NC * _NS
_O = NB * NH * LQ
_OPW = _O // _NW
_CH = 16
_NCHUNK = _OPW // _CH
_NT = 4 * NL * NP


def _sc_body(val_hbm, idx_hbm, w_hbm, out_hbm, val_sh, idx_v, w_v, rows_v,
             out_v, sem_g, sem_t):
    cid = lax.axis_index("c")
    wid = cid * _NS + lax.axis_index("s")
    base = wid * _OPW
    cbase = wid * _NCHUNK

    def fire_taps(ci, tb):
        row = cbase + ci
        pltpu.async_copy(idx_hbm.at[pl.ds(row, 1)], idx_v.at[tb], sem_t)
        pltpu.async_copy(w_hbm.at[pl.ds(row, 1)], w_v.at[tb], sem_t)

    def drain_taps(tb):
        pltpu.make_async_copy(
            idx_hbm.at[pl.ds(0, 1)], idx_v.at[tb], sem_t).wait()
        pltpu.make_async_copy(
            w_hbm.at[pl.ds(0, 1)], w_v.at[tb], sem_t).wait()

    def fire_gathers(tb, rb):
        for j in range(_CH):
            pltpu.async_copy(
                val_sh.at[idx_v.at[tb].at[0].at[pl.ds(j * _NT, _NT)]],
                rows_v.at[rb].at[pl.ds(j * _NT, _NT)], sem_g)

    @pl.when(lax.axis_index("s") == 0)
    def _():
        pltpu.sync_copy(val_hbm.at[pl.ds(cid * (LEN_IN * NH), LEN_IN * NH)],
                        val_sh)
    plsc.subcore_barrier()

    pltpu.sync_copy(idx_hbm.at[pl.ds(cbase, 1)], idx_v.at[0])
    pltpu.sync_copy(w_hbm.at[pl.ds(cbase, 1)], w_v.at[0])
    fire_gathers(0, 0)
    fire_taps(1, 1)

    def chunk(c, carry):
        rb = lax.rem(c, 2)
        tb_c = lax.rem(c, 3)
        tb_n = lax.rem(c + 1, 3)
        tb_p = lax.rem(c + 2, 3)
        pltpu.make_async_copy(
            val_hbm.at[pl.ds(0, _CH * _NT)], rows_v.at[rb], sem_g).wait()

        @pl.when(c + 1 < _NCHUNK)
        def _():
            drain_taps(tb_n)
            fire_gathers(tb_n, 1 - rb)

        for j in range(_CH):
            def grp(g, carry):
                m0, m1 = carry
                wvp = w_v[tb_c, 0, pl.ds(j * _NT + g * 16, 16)]
                r0 = j * _NT + g * 16
                b = [jnp.zeros((32,), jnp.bfloat16) for _ in range(4)]
                for k in range(16):
                    wt = plsc.bitcast(
                        jnp.full((16,), wvp[k], dtype=jnp.int32),
                        jnp.bfloat16)
                    b[k % 4] = b[k % 4] + wt * rows_v[rb, r0 + k]
                s = (b[0] + b[1]) + (b[2] + b[3])
                e, o = plsc.unpack(s, format=plsc.PackFormat.INTERLEAVED)
                return (m0 + e, m1 + o)

            m0, m1 = lax.fori_loop(
                0, _NT // 16, grp,
                (jnp.zeros((16,), jnp.float32), jnp.zeros((16,), jnp.float32)))
            out_v[j, pl.ds(0, 16)] = m0
            out_v[j, pl.ds(16, 16)] = m1
        pltpu.sync_copy(out_v, out_hbm.at[pl.ds(base + c * _CH, _CH)])

        @pl.when(c + 2 < _NCHUNK)
        def _():
            fire_taps(c + 2, tb_p)

        return carry

    lax.fori_loop(0, _NCHUNK, chunk, 0)


@functools.partial(jax.jit)
def _sc_gather(val_flat, idx, w):
    mesh = plsc.VectorSubcoreMesh(core_axis_name="c", subcore_axis_name="s")
    return pl.kernel(
        _sc_body,
        out_type=jax.ShapeDtypeStruct((_O, D), jnp.float32),
        mesh=mesh,
        scratch_types=[
            pltpu.VMEM_SHARED((LEN_IN * NH, D), jnp.bfloat16),
            pltpu.VMEM((3, 1, _CH * _NT), jnp.int32),
            pltpu.VMEM((3, 1, _CH * _NT), jnp.int32),
            pltpu.VMEM((2, _CH * _NT, D), jnp.bfloat16),
            pltpu.VMEM((_CH, D), jnp.float32),
            pltpu.SemaphoreType.DMA,
            pltpu.SemaphoreType.DMA,
        ],
        compiler_params=pltpu.CompilerParams(use_tc_tiling_on_sc=False,
                                             needs_layout_passes=False),
    )(val_flat, idx, w)


def _out_body(core_ref, tgt_ref, wo_ref, bo_ref, g_ref, b_ref,
              wfc1_ref, bfc1_ref, wfc2_ref, bfc2_ref, out_ref):
    def dot(a, w):
        return lax.dot_general(a, w, (((1,), (1,)), ((), ())),
                               preferred_element_type=jnp.float32)

    acc = jnp.broadcast_to(bo_ref[0], (LQ, C))
    for hh in range(NH):
        acc = acc + dot(core_ref[0, hh], wo_ref[:, hh * D:(hh + 1) * D])
    tgt2 = tgt_ref[0] + acc
    mu = jnp.mean(tgt2, axis=-1, keepdims=True)
    var = jnp.mean((tgt2 - mu) ** 2, axis=-1, keepdims=True)
    tn = (tgt2 - mu) / jnp.sqrt(var + 1e-5) * g_ref[0] + b_ref[0]
    hm = dot(tn, wfc1_ref[...]) + bfc1_ref[0]
    hm = 0.5 * hm * (1.0 + lax.erf(hm * (1.0 / math.sqrt(2.0))))
    out_ref[0] = tgt2 + dot(hm, wfc2_ref[...]) + bfc2_ref[0]


def _out_call(core, tgt, wo, bo, g, b, wfc1, bfc1, wfc2, bfc2):
    return pl.pallas_call(
        _out_body,
        grid=(NB,),
        in_specs=[
            pl.BlockSpec((1, NH, LQ, D), lambda n: (n, 0, 0, 0)),
            pl.BlockSpec((1, LQ, C), lambda n: (n, 0, 0)),
            pl.BlockSpec((C, C), lambda n: (0, 0)),
            pl.BlockSpec((1, C), lambda n: (0, 0)),
            pl.BlockSpec((1, C), lambda n: (0, 0)),
            pl.BlockSpec((1, C), lambda n: (0, 0)),
            pl.BlockSpec((4 * C, C), lambda n: (0, 0)),
            pl.BlockSpec((1, 4 * C), lambda n: (0, 0)),
            pl.BlockSpec((C, 4 * C), lambda n: (0, 0)),
            pl.BlockSpec((1, C), lambda n: (0, 0)),
        ],
        out_specs=pl.BlockSpec((1, LQ, C), lambda n: (n, 0, 0)),
        out_shape=jax.ShapeDtypeStruct((NB, LQ, C), jnp.float32),
    )(core, tgt, wo, bo.reshape(1, C), g.reshape(1, C), b.reshape(1, C),
      wfc1, bfc1.reshape(1, 4 * C), wfc2, bfc2.reshape(1, C))


def kernel(tgt, reference_points, src, src_spatial_shapes, level_start_index,
           norm1_g, norm1_b, norm2_g, norm2_b, W_so, b_so, W_aw, b_aw,
           W_v, b_v, W_o, b_o, W_fc1, b_fc1, W_fc2, b_fc2):
    value = _value_call(src, norm1_g, norm1_b, W_v, b_v)
    val_flat = value.reshape(NB * LEN_IN * NH, D)

    rx = jnp.broadcast_to(reference_points[:, :, None, :, None, 0],
                          (NB, LQ, NH, NL, NP)).reshape(NB, LQ, NH * NL * NP)
    ry = jnp.broadcast_to(reference_points[:, :, None, :, None, 1],
                          (NB, LQ, NH, NL, NP)).reshape(NB, LQ, NH * NL * NP)
    wso_r = W_so.reshape(NH, NL, NP, 2, C)
    bso_r = b_so.reshape(NH, NL, NP, 2)
    wsox = wso_r[:, :, :, 0, :].reshape(NH * NL * NP, C)
    wsoy = wso_r[:, :, :, 1, :].reshape(NH * NL * NP, C)
    bsox = bso_r[:, :, :, 0].reshape(1, NH * NL * NP)
    bsoy = bso_r[:, :, :, 1].reshape(1, NH * NL * NP)
    waw = W_aw.reshape(NH * NL * NP, C)
    baw = b_aw.reshape(1, NH * NL * NP)

    idx, w = _tap_call(tgt, rx, ry, wsox, wsoy, waw, bsox, bsoy, baw,
                       norm1_g, norm1_b)
    core = _sc_gather(val_flat, idx.reshape(_O // _CH, _CH * _NT),
                      w.reshape(_O // _CH, _CH * _NT))
    core4 = core.reshape(NB, NH, LQ, D)
    wo_perm = W_o.reshape(C, NH, D // 2, 2).transpose(0, 1, 3, 2).reshape(C, C)
    return _out_call(core4, tgt, wo_perm, b_o, norm2_g, norm2_b,
                     W_fc1, b_fc1, W_fc2, b_fc2)
